# baseline (speedup 1.0000x reference)
import jax
import jax.numpy as jnp
from jax.experimental import pallas as pl

_F_BLK = 256
_POS_ROWS = _F_BLK // 128


def _add_body(x_ref, pos_ref, o_ref):
    o_ref[...] = x_ref[...] + jnp.transpose(pos_ref[0], (1, 0))


def kernel(x, pos_table):
    batch, maxlen, embed = x.shape
    flat = maxlen * embed
    xt = x.transpose(1, 2, 0).reshape(flat, batch)
    post = pos_table.reshape(flat // _F_BLK, 1, _F_BLK)

    grid = (flat // _F_BLK,)
    out_t = pl.pallas_call(
        _add_body,
        grid=grid,
        in_specs=[
            pl.BlockSpec((_F_BLK, batch), lambda i: (i, 0)),
            pl.BlockSpec((1, 1, _F_BLK), lambda i: (i, 0, 0)),
        ],
        out_specs=pl.BlockSpec((_F_BLK, batch), lambda i: (i, 0)),
        out_shape=jax.ShapeDtypeStruct((flat, batch), x.dtype),
    )(xt, post)
    return out_t.reshape(maxlen, embed, batch).transpose(2, 0, 1)


# speedup vs baseline: 1.0204x; 1.0204x over previous
import jax
import jax.numpy as jnp
from jax.experimental import pallas as pl

_F_BLK = 800
_POS_ROWS = _F_BLK // 128


def _add_body(x_ref, pos_ref, o_ref):
    o_ref[...] = x_ref[...] + jnp.transpose(pos_ref[0], (1, 0))


def kernel(x, pos_table):
    batch, maxlen, embed = x.shape
    flat = maxlen * embed
    xt = x.transpose(1, 2, 0).reshape(flat, batch)
    post = pos_table.reshape(flat // _F_BLK, 1, _F_BLK)

    grid = (flat // _F_BLK,)
    out_t = pl.pallas_call(
        _add_body,
        grid=grid,
        in_specs=[
            pl.BlockSpec((_F_BLK, batch), lambda i: (i, 0)),
            pl.BlockSpec((1, 1, _F_BLK), lambda i: (i, 0, 0)),
        ],
        out_specs=pl.BlockSpec((_F_BLK, batch), lambda i: (i, 0)),
        out_shape=jax.ShapeDtypeStruct((flat, batch), x.dtype),
    )(xt, post)
    return out_t.reshape(maxlen, embed, batch).transpose(2, 0, 1)
